# baseline (speedup 1.0000x reference)
import jax
import jax.numpy as jnp
from jax.experimental import pallas as pl


def _tanh_block(x_ref, o_ref):
    o_ref[...] = jnp.tanh(x_ref[...])


def kernel(input, images):
    b = input.shape[0]
    n = images.shape[0]
    if b < n:
        images = images[:b]
    shape = images.shape
    h, w = shape[-2], shape[-1]
    rows = images.size // (h * w)
    block = 4
    grid = rows // block
    x = images.reshape(rows, h, w)
    out = pl.pallas_call(
        _tanh_block,
        out_shape=jax.ShapeDtypeStruct((rows, h, w), jnp.float32),
        grid=(grid,),
        in_specs=[pl.BlockSpec((block, h, w), lambda i: (i, 0, 0))],
        out_specs=pl.BlockSpec((block, h, w), lambda i: (i, 0, 0)),
    )(x)
    return out.reshape(shape)


# speedup vs baseline: 1.1037x; 1.1037x over previous
import jax
import jax.numpy as jnp
from jax.experimental import pallas as pl


def _tanh_block(x_ref, o_ref):
    o_ref[...] = jnp.tanh(x_ref[...])


def kernel(input, images):
    b = input.shape[0]
    n = images.shape[0]
    if b < n:
        images = images[:b]
    shape = images.shape
    h, w = shape[-2], shape[-1]
    rows = images.size // (h * w)
    block = 16
    grid = rows // block
    x = images.reshape(rows, h, w)
    out = pl.pallas_call(
        _tanh_block,
        out_shape=jax.ShapeDtypeStruct((rows, h, w), jnp.float32),
        grid=(grid,),
        in_specs=[pl.BlockSpec((block, h, w), lambda i: (i, 0, 0))],
        out_specs=pl.BlockSpec((block, h, w), lambda i: (i, 0, 0)),
    )(x)
    return out.reshape(shape)


# speedup vs baseline: 1.1176x; 1.0125x over previous
import jax
import jax.numpy as jnp
from jax.experimental import pallas as pl


def _tanh_block(x_ref, o_ref):
    o_ref[...] = jnp.tanh(x_ref[...])


def kernel(input, images):
    b = input.shape[0]
    n = images.shape[0]
    if b < n:
        images = images[:b]
    shape = images.shape
    h, w = shape[-2], shape[-1]
    rows = images.size // (h * w)
    block = 24
    grid = rows // block
    x = images.reshape(rows, h, w)
    out = pl.pallas_call(
        _tanh_block,
        out_shape=jax.ShapeDtypeStruct((rows, h, w), jnp.float32),
        grid=(grid,),
        in_specs=[pl.BlockSpec((block, h, w), lambda i: (i, 0, 0))],
        out_specs=pl.BlockSpec((block, h, w), lambda i: (i, 0, 0)),
    )(x)
    return out.reshape(shape)
